# Initial kernel scaffold; baseline (speedup 1.0000x reference)
import jax
import jax.numpy as jnp
from jax.experimental import pallas as pl


def kernel(x, patch, pos):
    raise NotImplementedError("write your pallas kernel here")


# speedup vs baseline: 4.9896x; 4.9896x over previous
import jax
import jax.numpy as jnp
from jax.experimental import pallas as pl
from jax.experimental.pallas import tpu as pltpu

ISIZE = 224
PSIZE = 30
WIN = 40


def _place_kernel(pos_ref, x_ref, patch_pad_ref, out_ref):
    b = pl.program_id(0)
    py = pos_ref[b, 0]
    px = pos_ref[b, 1]
    out_ref[0] = x_ref[0]
    ry = jnp.minimum((py // 8) * 8, ISIZE - WIN)
    ry = pl.multiple_of(ry, 8)
    dy = py - ry
    tile = patch_pad_ref[0]
    tile = pltpu.roll(tile, px, axis=2)
    tile = pltpu.roll(tile, dy, axis=1)
    win = x_ref[0, :, pl.ds(ry, WIN), :]
    out_ref[0, :, pl.ds(ry, WIN), :] = win + tile


def kernel(x, patch, pos):
    B = x.shape[0]
    patch_pad = jnp.zeros((1, 3, WIN, ISIZE), dtype=patch.dtype)
    patch_pad = jax.lax.dynamic_update_slice(patch_pad, patch, (0, 0, 0, 0))
    grid_spec = pltpu.PrefetchScalarGridSpec(
        num_scalar_prefetch=1,
        grid=(B,),
        in_specs=[
            pl.BlockSpec((1, 3, ISIZE, ISIZE), lambda b, pos_ref: (b, 0, 0, 0)),
            pl.BlockSpec((1, 3, WIN, ISIZE), lambda b, pos_ref: (0, 0, 0, 0)),
        ],
        out_specs=pl.BlockSpec((1, 3, ISIZE, ISIZE), lambda b, pos_ref: (b, 0, 0, 0)),
    )
    return pl.pallas_call(
        _place_kernel,
        grid_spec=grid_spec,
        out_shape=jax.ShapeDtypeStruct(x.shape, x.dtype),
    )(pos, x, patch_pad)


# speedup vs baseline: 6.1301x; 1.2286x over previous
import jax
import jax.numpy as jnp
from jax.experimental import pallas as pl
from jax.experimental.pallas import tpu as pltpu

ISIZE = 224
PSIZE = 30
WIN = 40


BS = 4


def _place_kernel(pos_ref, x_ref, patch_pad_ref, out_ref):
    g = pl.program_id(0)
    out_ref[...] = x_ref[...]
    for i in range(BS):
        b = g * BS + i
        py = pos_ref[b, 0]
        px = pos_ref[b, 1]
        ry = jnp.minimum((py // 8) * 8, ISIZE - WIN)
        ry = pl.multiple_of(ry, 8)
        dy = py - ry
        tile = patch_pad_ref[0]
        tile = pltpu.roll(tile, px, axis=2)
        tile = pltpu.roll(tile, dy, axis=1)
        win = x_ref[i, :, pl.ds(ry, WIN), :]
        out_ref[i, :, pl.ds(ry, WIN), :] = win + tile


def kernel(x, patch, pos):
    B = x.shape[0]
    patch_pad = jnp.zeros((1, 3, WIN, ISIZE), dtype=patch.dtype)
    patch_pad = jax.lax.dynamic_update_slice(patch_pad, patch, (0, 0, 0, 0))
    grid_spec = pltpu.PrefetchScalarGridSpec(
        num_scalar_prefetch=1,
        grid=(B // BS,),
        in_specs=[
            pl.BlockSpec((BS, 3, ISIZE, ISIZE), lambda b, pos_ref: (b, 0, 0, 0)),
            pl.BlockSpec((1, 3, WIN, ISIZE), lambda b, pos_ref: (0, 0, 0, 0)),
        ],
        out_specs=pl.BlockSpec((BS, 3, ISIZE, ISIZE), lambda b, pos_ref: (b, 0, 0, 0)),
    )
    return pl.pallas_call(
        _place_kernel,
        grid_spec=grid_spec,
        out_shape=jax.ShapeDtypeStruct(x.shape, x.dtype),
    )(pos, x, patch_pad)


# speedup vs baseline: 6.2313x; 1.0165x over previous
import jax
import jax.numpy as jnp
from jax.experimental import pallas as pl
from jax.experimental.pallas import tpu as pltpu

ISIZE = 224
PSIZE = 30
WIN = 40


BS = 8


def _place_kernel(pos_ref, x_ref, patch_pad_ref, out_ref):
    g = pl.program_id(0)
    out_ref[...] = x_ref[...]
    for i in range(BS):
        b = g * BS + i
        py = pos_ref[b, 0]
        px = pos_ref[b, 1]
        ry = jnp.minimum((py // 8) * 8, ISIZE - WIN)
        ry = pl.multiple_of(ry, 8)
        dy = py - ry
        tile = patch_pad_ref[0]
        tile = pltpu.roll(tile, px, axis=2)
        tile = pltpu.roll(tile, dy, axis=1)
        win = x_ref[i, :, pl.ds(ry, WIN), :]
        out_ref[i, :, pl.ds(ry, WIN), :] = win + tile


def kernel(x, patch, pos):
    B = x.shape[0]
    patch_pad = jnp.zeros((1, 3, WIN, ISIZE), dtype=patch.dtype)
    patch_pad = jax.lax.dynamic_update_slice(patch_pad, patch, (0, 0, 0, 0))
    grid_spec = pltpu.PrefetchScalarGridSpec(
        num_scalar_prefetch=1,
        grid=(B // BS,),
        in_specs=[
            pl.BlockSpec((BS, 3, ISIZE, ISIZE), lambda b, pos_ref: (b, 0, 0, 0)),
            pl.BlockSpec((1, 3, WIN, ISIZE), lambda b, pos_ref: (0, 0, 0, 0)),
        ],
        out_specs=pl.BlockSpec((BS, 3, ISIZE, ISIZE), lambda b, pos_ref: (b, 0, 0, 0)),
    )
    return pl.pallas_call(
        _place_kernel,
        grid_spec=grid_spec,
        out_shape=jax.ShapeDtypeStruct(x.shape, x.dtype),
    )(pos, x, patch_pad)
